# Initial kernel scaffold; baseline (speedup 1.0000x reference)
import jax
import jax.numpy as jnp
from jax.experimental import pallas as pl


def kernel(x, table):
    raise NotImplementedError("write your pallas kernel here")


# speedup vs baseline: 1.1116x; 1.1116x over previous
import functools

import jax
import jax.numpy as jnp
from jax import lax
from jax.experimental import pallas as pl
from jax.experimental.pallas import tpu as pltpu
from jax.experimental.pallas import tpu_sc as plsc

EMBED_DIM = 32
NUM_CORES = 2
NUM_SUBCORES = 16
NUM_WORKERS = NUM_CORES * NUM_SUBCORES


def _make_gather(batch: int, vocab: int, chunk: int):
    assert batch % NUM_WORKERS == 0
    b_per_w = batch // NUM_WORKERS
    assert b_per_w % chunk == 0
    n_chunks = b_per_w // chunk

    mesh = plsc.VectorSubcoreMesh(core_axis_name="c", subcore_axis_name="s")

    @functools.partial(
        pl.kernel,
        mesh=mesh,
        compiler_params=pltpu.CompilerParams(use_tc_tiling_on_sc=False),
        out_type=jax.ShapeDtypeStruct((batch, EMBED_DIM), jnp.float32),
        scratch_types=[
            pltpu.VMEM((b_per_w,), jnp.int32),
            pltpu.VMEM((chunk, EMBED_DIM), jnp.float32),
            pltpu.VMEM((chunk, EMBED_DIM), jnp.float32),
            pltpu.SemaphoreType.DMA,
            pltpu.SemaphoreType.DMA,
        ],
    )
    def gather_kernel(idx_hbm, table_hbm, out_hbm, idx_v, rows0, rows1, sem0, sem1):
        wid = lax.axis_index("s") * NUM_CORES + lax.axis_index("c")
        base = wid * b_per_w
        pltpu.sync_copy(idx_hbm.at[pl.ds(base, b_per_w)], idx_v)

        rows = (rows0, rows1)
        sems = (sem0, sem1)

        def start(i, buf):
            pltpu.async_copy(
                table_hbm.at[idx_v.at[pl.ds(i * chunk, chunk)]],
                rows[buf],
                sems[buf],
            )

        start(0, 0)
        for i in range(n_chunks):
            buf = i % 2
            if i + 1 < n_chunks:
                start(i + 1, 1 - buf)
            pltpu.make_async_copy(table_hbm.at[idx_v.at[pl.ds(i * chunk, chunk)]],
                                  rows[buf], sems[buf]).wait()
            pltpu.sync_copy(rows[buf], out_hbm.at[pl.ds(base + i * chunk, chunk)])

    return gather_kernel


@jax.jit
def kernel(x, table):
    batch, hist = x.shape
    flat = x.reshape(-1).astype(jnp.int32)
    gather = _make_gather(flat.shape[0], table.shape[0], chunk=1280)
    out = gather(flat, table)
    return out.reshape(batch, hist, EMBED_DIM)


# speedup vs baseline: 1.8020x; 1.6212x over previous
import functools

import jax
import jax.numpy as jnp
from jax import lax
from jax.experimental import pallas as pl
from jax.experimental.pallas import tpu as pltpu
from jax.experimental.pallas import tpu_sc as plsc

EMBED_DIM = 32
NUM_CORES = 2
NUM_SUBCORES = 16
NUM_WORKERS = NUM_CORES * NUM_SUBCORES


def _make_gather(batch: int, hist: int, vocab: int, chunk_rows: int):
    assert batch % NUM_WORKERS == 0
    rows_per_w = batch // NUM_WORKERS
    assert rows_per_w % chunk_rows == 0
    n_chunks = rows_per_w // chunk_rows

    mesh = plsc.VectorSubcoreMesh(core_axis_name="c", subcore_axis_name="s")

    @functools.partial(
        pl.kernel,
        mesh=mesh,
        compiler_params=pltpu.CompilerParams(use_tc_tiling_on_sc=False),
        out_type=jax.ShapeDtypeStruct((batch, hist, EMBED_DIM), jnp.float32),
        scratch_types=[
            pltpu.VMEM((rows_per_w, hist), jnp.int32),
            pltpu.VMEM((chunk_rows, hist, EMBED_DIM), jnp.float32),
            pltpu.VMEM((chunk_rows, hist, EMBED_DIM), jnp.float32),
            pltpu.SemaphoreType.DMA,
            pltpu.SemaphoreType.DMA,
        ],
    )
    def gather_kernel(x_hbm, table_hbm, out_hbm, idx_v, rows0, rows1, sem0, sem1):
        wid = lax.axis_index("s") * NUM_CORES + lax.axis_index("c")
        base = wid * rows_per_w
        pltpu.sync_copy(x_hbm.at[pl.ds(base, rows_per_w)], idx_v)

        rows = (rows0, rows1)
        sems = (sem0, sem1)

        def start(i, buf):
            for j in range(chunk_rows):
                pltpu.async_copy(
                    table_hbm.at[idx_v.at[i * chunk_rows + j]],
                    rows[buf].at[j],
                    sems[buf],
                )

        def drain(i, buf):
            for j in range(chunk_rows):
                pltpu.make_async_copy(
                    table_hbm.at[idx_v.at[i * chunk_rows + j]],
                    rows[buf].at[j],
                    sems[buf],
                ).wait()

        start(0, 0)

        @pl.loop(0, n_chunks // 2)
        def _body(p):
            i = p * 2
            start(i + 1, 1)
            drain(i, 0)
            pltpu.sync_copy(rows[0],
                            out_hbm.at[pl.ds(base + i * chunk_rows, chunk_rows)])
            @pl.when(i + 2 < n_chunks)
            def _():
                start(i + 2, 0)
            drain(i + 1, 1)
            pltpu.sync_copy(rows[1],
                            out_hbm.at[pl.ds(base + (i + 1) * chunk_rows,
                                             chunk_rows)])

    return gather_kernel


@jax.jit
def kernel(x, table):
    batch, hist = x.shape
    gather = _make_gather(batch, hist, table.shape[0], chunk_rows=8)
    return gather(x.astype(jnp.int32), table)
